# baseline (speedup 1.0000x reference)
import functools

import jax
import jax.numpy as jnp
from jax import lax
from jax.experimental import pallas as pl
from jax.experimental.pallas import tpu as pltpu
from jax.experimental.pallas import tpu_sc as plsc

_B, _C, _S = 256, 768, 196
_NC = 10
_LANE = 16

_SBK = 7
_NRD = 10
_SC_S = _SBK * _NRD
_TC_S = _S - _SC_S
_NW = 32
_BPW = _B // _NW

_SB = 14
_NSTEP = _TC_S // _SB


def _sc_body(f_hbm, out_hbm, buf0, buf1, acc, sem0, sem1):
    cid = lax.axis_index("c")
    sid = lax.axis_index("s")
    wid = sid * 2 + cid
    base = wid * _BPW

    def dma(r, buf, sem):
        return pltpu.make_async_copy(
            f_hbm.at[pl.ds(_TC_S + r * _SBK, _SBK), pl.ds(base, _BPW), :],
            buf, sem)

    def zbody(j, _):
        for r in range(_BPW):
            acc[r, pl.ds(j * _LANE, _LANE)] = jnp.zeros((_LANE,), jnp.float32)
        return 0
    lax.fori_loop(0, _C // _LANE, zbody, 0)

    dma(0, buf0, sem0).start()

    def accumulate(buf):
        def jbody(j, _):
            js = pl.ds(j * _LANE, _LANE)
            for r in range(_BPW):
                x = buf[0, r, js]
                for s in range(1, _SBK):
                    x = x + buf[s, r, js]
                plsc.addupdate(acc.at[r, js], x)
            return 0
        lax.fori_loop(0, _C // _LANE, jbody, 0)

    def outer(i2, _):
        r0 = i2 * 2
        for k in range(2):
            buf = (buf0, buf1)[k]
            sem = (sem0, sem1)[k]
            obuf = (buf1, buf0)[k]
            osem = (sem1, sem0)[k]
            r = r0 + k

            @pl.when(r + 1 < _NRD)
            def _prefetch():
                dma(r + 1, obuf, osem).start()

            dma(r, buf, sem).wait()
            accumulate(buf)
        return 0

    lax.fori_loop(0, _NRD // 2, outer, 0)

    pltpu.sync_copy(acc, out_hbm.at[pl.ds(base, _BPW), :])


_sc_pool = functools.partial(
    pl.kernel,
    out_type=jax.ShapeDtypeStruct((_B, _C), jnp.float32),
    mesh=plsc.VectorSubcoreMesh(core_axis_name="c", subcore_axis_name="s"),
    scratch_types=[
        pltpu.VMEM((_SBK, _BPW, _C), jnp.float32),
        pltpu.VMEM((_SBK, _BPW, _C), jnp.float32),
        pltpu.VMEM((_BPW, _C), jnp.float32),
        pltpu.SemaphoreType.DMA,
        pltpu.SemaphoreType.DMA,
    ],
    cost_estimate=pl.CostEstimate(
        flops=_SC_S * _B * _C,
        bytes_accessed=_SC_S * _B * _C * 4,
        transcendentals=0,
    ),
    compiler_params=pltpu.CompilerParams(skip_device_barrier=True),
)(_sc_body)


def _tc_pool_body(f_ref, o_ref, acc_ref):
    i = pl.program_id(0)
    partial = jnp.sum(f_ref[...], axis=0)

    @pl.when(i == 0)
    def _init():
        acc_ref[...] = partial

    @pl.when(i > 0)
    def _acc():
        acc_ref[...] += partial

    @pl.when(i == _NSTEP - 1)
    def _fin():
        o_ref[...] = acc_ref[...]


def _tc_head(p_ref, q_ref, w_ref, b_ref, o_ref):
    pooled = (p_ref[...] + q_ref[...]) * (1.0 / _S)
    o_ref[...] = jax.lax.dot_general(
        pooled, w_ref[...], (((1,), (1,)), ((), ())),
        preferred_element_type=jnp.float32) + b_ref[...]


def kernel(features, W, b):
    f3 = features.transpose(2, 3, 0, 1).reshape(_S, _B, _C)

    sums_sc = _sc_pool(f3)

    sums_tc = pl.pallas_call(
        _tc_pool_body,
        grid=(_NSTEP,),
        in_specs=[pl.BlockSpec((_SB, _B, _C), lambda i: (i, 0, 0))],
        out_specs=pl.BlockSpec((_B, _C), lambda i: (0, 0)),
        out_shape=jax.ShapeDtypeStruct((_B, _C), jnp.float32),
        scratch_shapes=[pltpu.VMEM((_B, _C), jnp.float32)],
        cost_estimate=pl.CostEstimate(
            flops=_TC_S * _B * _C,
            bytes_accessed=_TC_S * _B * _C * 4,
            transcendentals=0,
        ),
        compiler_params=pltpu.CompilerParams(skip_device_barrier=True),
    )(f3)

    out = pl.pallas_call(
        _tc_head,
        in_specs=[
            pl.BlockSpec((_B, _C), lambda: (0, 0)),
            pl.BlockSpec((_B, _C), lambda: (0, 0)),
            pl.BlockSpec((_NC, _C), lambda: (0, 0)),
            pl.BlockSpec((1, _NC), lambda: (0, 0)),
        ],
        out_specs=pl.BlockSpec((_B, _NC), lambda: (0, 0)),
        out_shape=jax.ShapeDtypeStruct((_B, _NC), jnp.float32),
    )(sums_tc, sums_sc, W, b.reshape(1, _NC))
    return out


# speedup vs baseline: 1.4216x; 1.4216x over previous
import jax
import jax.numpy as jnp
from jax.experimental import pallas as pl
from jax.experimental.pallas import tpu as pltpu

_B, _C, _S = 256, 768, 196
_NC = 10
_SB = 7
_NSTEP = _S // _SB


def _body(f_ref, w_ref, b_ref, o_ref, acc_ref):
    i = pl.program_id(0)
    partial = jnp.sum(f_ref[...], axis=0)

    @pl.when(i == 0)
    def _init():
        acc_ref[...] = partial

    @pl.when(i > 0)
    def _acc():
        acc_ref[...] += partial

    @pl.when(i == _NSTEP - 1)
    def _fin():
        pooled = acc_ref[...] * (1.0 / _S)
        o_ref[...] = jax.lax.dot_general(
            pooled, w_ref[...], (((1,), (1,)), ((), ())),
            preferred_element_type=jnp.float32) + b_ref[...]


def kernel(features, W, b):
    f3 = features.transpose(2, 3, 0, 1).reshape(_S, _B, _C)
    out = pl.pallas_call(
        _body,
        grid=(_NSTEP,),
        in_specs=[
            pl.BlockSpec((_SB, _B, _C), lambda i: (i, 0, 0)),
            pl.BlockSpec((_NC, _C), lambda i: (0, 0)),
            pl.BlockSpec((1, _NC), lambda i: (0, 0)),
        ],
        out_specs=pl.BlockSpec((_B, _NC), lambda i: (0, 0)),
        out_shape=jax.ShapeDtypeStruct((_B, _NC), jnp.float32),
        scratch_shapes=[pltpu.VMEM((_B, _C), jnp.float32)],
    )(f3, W, b.reshape(1, _NC))
    return out
